# baseline (speedup 1.0000x reference)
import jax
import jax.numpy as jnp
from jax.experimental import pallas as pl
from jax.experimental.pallas import tpu as pltpu

S_BLK = 4096
N_CHUNK = 32


def _attn_pool_kernel(x_ref, xb_ref, wt_ref, b_ref, o_ref, num_ref, den_ref):
    j = pl.program_id(1)
    nj = pl.num_programs(1)
    C = S_BLK // N_CHUNK
    nums = []
    dens = []
    for c in range(N_CHUNK):
        xc = x_ref[0][c * C : (c + 1) * C, :]
        xbc = xb_ref[0][c * C : (c + 1) * C, :]
        e = jnp.tanh(
            jnp.dot(xbc, wt_ref[...], preferred_element_type=jnp.float32)
            + b_ref[...]
        )
        p = jnp.exp(e)
        nums.append(jnp.sum(p * xc, axis=0, keepdims=True))
        dens.append(jnp.sum(p, axis=0, keepdims=True))
    num = nums[0]
    den = dens[0]
    for c in range(1, N_CHUNK):
        num = num + nums[c]
        den = den + dens[c]

    @pl.when(j == 0)
    def _():
        num_ref[...] = num
        den_ref[...] = den

    @pl.when(j > 0)
    def _():
        num_ref[...] = num_ref[...] + num
        den_ref[...] = den_ref[...] + den

    @pl.when(j == nj - 1)
    def _():
        o_ref[0] = num_ref[...] / den_ref[...]


def kernel(lstm_output, W, b):
    B, S, H = lstm_output.shape
    Wt = W.T.astype(jnp.bfloat16)
    xb = lstm_output.astype(jnp.bfloat16)
    b2 = b.reshape(1, H)
    grid = (B, S // S_BLK)
    return pl.pallas_call(
        _attn_pool_kernel,
        grid=grid,
        in_specs=[
            pl.BlockSpec((1, S_BLK, H), lambda i, j: (i, j, 0)),
            pl.BlockSpec((1, S_BLK, H), lambda i, j: (i, j, 0)),
            pl.BlockSpec((H, H), lambda i, j: (0, 0)),
            pl.BlockSpec((1, H), lambda i, j: (0, 0)),
        ],
        out_specs=pl.BlockSpec((1, 1, H), lambda i, j: (i, 0, 0)),
        out_shape=jax.ShapeDtypeStruct((B, 1, H), jnp.float32),
        scratch_shapes=[
            pltpu.VMEM((1, H), jnp.float32),
            pltpu.VMEM((1, H), jnp.float32),
        ],
        compiler_params=pltpu.CompilerParams(
            dimension_semantics=("parallel", "arbitrary"),
        ),
        name="attn_pool",
    )(lstm_output, xb, Wt, b2).reshape(B, H)


# speedup vs baseline: 1.7899x; 1.7899x over previous
import jax
import jax.numpy as jnp
from jax.experimental import pallas as pl
from jax.experimental.pallas import tpu as pltpu

S_BLK = 4096
N_CHUNK = 32


def _attn_pool_kernel(x_ref, wt_ref, b_ref, o_ref, num_ref, den_ref):
    j = pl.program_id(1)
    nj = pl.num_programs(1)
    C = S_BLK // N_CHUNK
    nums = []
    dens = []
    for c in range(N_CHUNK):
        xc = x_ref[0][c * C : (c + 1) * C, :]
        e = jnp.tanh(
            jnp.dot(xc, wt_ref[...], preferred_element_type=jnp.float32)
            + b_ref[...]
        )
        p = jnp.exp(e)
        nums.append(jnp.sum(p * xc, axis=0, keepdims=True))
        dens.append(jnp.sum(p, axis=0, keepdims=True))
    num = nums[0]
    den = dens[0]
    for c in range(1, N_CHUNK):
        num = num + nums[c]
        den = den + dens[c]

    @pl.when(j == 0)
    def _():
        num_ref[...] = num
        den_ref[...] = den

    @pl.when(j > 0)
    def _():
        num_ref[...] = num_ref[...] + num
        den_ref[...] = den_ref[...] + den

    @pl.when(j == nj - 1)
    def _():
        o_ref[0] = num_ref[...] / den_ref[...]


def kernel(lstm_output, W, b):
    B, S, H = lstm_output.shape
    Wt = W.T
    b2 = b.reshape(1, H)
    grid = (B, S // S_BLK)
    return pl.pallas_call(
        _attn_pool_kernel,
        grid=grid,
        in_specs=[
            pl.BlockSpec((1, S_BLK, H), lambda i, j: (i, j, 0)),
            pl.BlockSpec((H, H), lambda i, j: (0, 0)),
            pl.BlockSpec((1, H), lambda i, j: (0, 0)),
        ],
        out_specs=pl.BlockSpec((1, 1, H), lambda i, j: (i, 0, 0)),
        out_shape=jax.ShapeDtypeStruct((B, 1, H), jnp.float32),
        scratch_shapes=[
            pltpu.VMEM((1, H), jnp.float32),
            pltpu.VMEM((1, H), jnp.float32),
        ],
        compiler_params=pltpu.CompilerParams(
            dimension_semantics=("parallel", "arbitrary"),
        ),
        name="attn_pool",
    )(lstm_output, Wt, b2).reshape(B, H)
